# baseline (speedup 1.0000x reference)
import functools

import jax
import jax.numpy as jnp
from jax import lax
from jax.experimental import pallas as pl
from jax.experimental.pallas import tpu as pltpu
from jax.experimental.pallas import tpu_sc as plsc

HEIGHT = 480
K = 16
OUT = 64
R2 = (5.0 / HEIGHT) ** 2

B = 4
N = 4096
G = 64
NC = G * G
BIG = 1 << 30
L = 16
QPT = N // 8


def _sc_ball_kernel(px_hbm, py_hbm, zeros_hbm, emb_hbm, start_hbm, order_hbm,
                    pxv, pyv, cellv, subh, startv, orderv, embv, histv, totv,
                    sem1, sem2):
    c = lax.axis_index("core")
    s = lax.axis_index("subcore")
    lb = s // 8
    b = 2 * c + lb
    qs = (s % 8) * QPT

    lanes = lax.broadcasted_iota(jnp.int32, (L,), 0)
    ones = jnp.ones((L,), jnp.int32)

    cpx = pltpu.async_copy(px_hbm.at[b], pxv, sem1)
    cpy = pltpu.async_copy(py_hbm.at[b], pyv, sem2)
    cpx.wait()
    cpy.wait()

    @pl.when(s % 8 == 0)
    def _build():
        @pl.loop(0, N // L)
        def _cells(t):
            i0 = t * L
            x = pxv[pl.ds(i0, L)]
            y = pyv[pl.ds(i0, L)]
            cx = jnp.minimum((x * G).astype(jnp.int32), G - 1)
            cy = jnp.minimum((y * G).astype(jnp.int32), G - 1)
            cellv[pl.ds(i0, L)] = cy * G + cx

        pltpu.sync_copy(zeros_hbm, subh)

        mlow = lanes < 8
        mhigh = jnp.logical_not(mlow)
        rowbase = (lanes % 8) * NC

        @pl.loop(0, N // L)
        def _hist(t):
            cells16 = cellv[pl.ds(t * L, L)]
            flat = rowbase + cells16
            plsc.addupdate_scatter(subh, [flat], ones, mask=mlow)
            plsc.addupdate_scatter(subh, [flat], ones, mask=mhigh)

        @pl.loop(0, NC // L)
        def _pa(t):
            c0 = t * L
            hist16 = subh[pl.ds(c0, L)]
            for l in range(1, 8):
                hist16 = hist16 + subh[pl.ds(l * NC + c0, L)]
            histv[pl.ds(c0, L)] = hist16
            totv[t] = jnp.sum(hist16)

        def _pb(t, carry):
            tv = totv[t]
            totv[t] = carry
            return carry + tv

        lax.fori_loop(0, NC // L, _pb, jnp.int32(0))

        @pl.loop(0, NC // L)
        def _pc(t):
            c0 = t * L
            hist16 = histv[pl.ds(c0, L)]
            incl = plsc.cumsum(hist16)
            start16 = incl - hist16 + totv[t]
            startv[pl.ds(c0, L)] = start16
            run = start16
            for l in range(8):
                v = subh[pl.ds(l * NC + c0, L)]
                subh[pl.ds(l * NC + c0, L)] = run
                run = run + v
        startv[pl.ds(NC, L)] = jnp.full((L,), N, jnp.int32)

        @pl.loop(0, N // L)
        def _scatter(t):
            i0 = t * L
            cells16 = cellv[pl.ds(i0, L)]
            flat = rowbase + cells16
            idxv = i0 + lanes
            pos1 = plsc.load_gather(subh, [flat], mask=mlow)
            plsc.addupdate_scatter(subh, [flat], ones, mask=mlow)
            plsc.store_scatter(orderv, [pos1], idxv, mask=mlow)
            pos2 = plsc.load_gather(subh, [flat], mask=mhigh)
            plsc.addupdate_scatter(subh, [flat], ones, mask=mhigh)
            plsc.store_scatter(orderv, [pos2], idxv, mask=mhigh)

        cps = pltpu.async_copy(startv, start_hbm.at[b], sem1)
        cpo = pltpu.async_copy(orderv, order_hbm.at[b], sem2)
        cps.wait()
        cpo.wait()

    plsc.subcore_barrier()

    cps = pltpu.async_copy(start_hbm.at[b], startv, sem1)
    cpo = pltpu.async_copy(order_hbm.at[b], orderv, sem2)
    cps.wait()
    cpo.wait()

    @pl.loop(0, QPT // L)
    def _group(g):
        q0 = qs + g * L
        qi = q0 + lanes
        qx = pxv[pl.ds(q0, L)]
        qy = pyv[pl.ds(q0, L)]
        cx = jnp.minimum((qx * G).astype(jnp.int32), G - 1)
        cy = jnp.minimum((qy * G).astype(jnp.int32), G - 1)
        c1 = jnp.maximum(cx - 1, 0)
        c2 = jnp.minimum(cx + 1, G - 1)

        slots = tuple(jnp.full((L,), BIG, jnp.int32) for _ in range(K))
        for dr in (-1, 0, 1):
            rr = cy + dr
            rvalid = (rr >= 0) & (rr < G)
            rrc = jnp.clip(rr, 0, G - 1)
            lo = plsc.load_gather(startv, [rrc * G + c1])
            hi = plsc.load_gather(startv, [rrc * G + c2 + 1])
            lo = jnp.clip(jnp.where(rvalid, lo, 0), 0, N)
            hi = jnp.clip(jnp.where(rvalid, hi, 0), 0, N)
            lenv = jnp.maximum(hi - lo, 0)
            maxlen = jnp.max(lenv)

            def _one(sl, t):
                active = t < lenv
                p = jnp.clip(jnp.where(active, lo + t, 0), 0, N - 1)
                j = jnp.clip(plsc.load_gather(orderv, [p]), 0, N - 1)
                x = plsc.load_gather(pxv, [j])
                y = plsc.load_gather(pyv, [j])
                dx = qx - x
                dy = qy - y
                qual = active & ((dx * dx + dy * dy) < R2)
                v = jnp.where(qual, j, BIG)
                ns = [jnp.minimum(sl[0], v)]
                for k in range(1, K):
                    ns.append(jnp.maximum(sl[k - 1], jnp.minimum(sl[k], v)))
                return tuple(ns)

            def _cand(t, sl):
                sl = _one(sl, 2 * t)
                return _one(sl, 2 * t + 1)

            slots = lax.fori_loop(0, (maxlen + 1) // 2, _cand, slots)

        rows = g * L + lanes
        for k in range(K):
            sk = slots[k]
            jk = jnp.where(sk < BIG, sk, qi)
            gx = plsc.load_gather(pxv, [jk])
            gy = plsc.load_gather(pyv, [jk])
            kcol = jnp.full((L,), k, jnp.int32)
            plsc.store_scatter(embv, [rows, kcol], qx - gx)
            plsc.store_scatter(embv, [rows, kcol + K], qy - gy)

    pltpu.sync_copy(embv, emb_hbm.at[b, pl.ds(qs, QPT)])


def _mlp_kernel(emb_ref, w1_ref, b1_ref, w2_ref, b2_ref, out_ref):
    e = emb_ref[...]
    h = lax.dot_general(e, w1_ref[...], (((1,), (0,)), ((), ())),
                        preferred_element_type=jnp.float32)
    h = jnp.maximum(h + b1_ref[...], 0.0)
    o = lax.dot_general(h, w2_ref[...], (((1,), (0,)), ((), ())),
                        preferred_element_type=jnp.float32)
    out_ref[...] = o + b2_ref[...]


_sc_ball = pl.kernel(
    _sc_ball_kernel,
    out_type=(jax.ShapeDtypeStruct((B, N, 2 * K), jnp.float32),
              jax.ShapeDtypeStruct((B, NC + L), jnp.int32),
              jax.ShapeDtypeStruct((B, N), jnp.int32)),
    mesh=plsc.VectorSubcoreMesh(core_axis_name="core",
                                subcore_axis_name="subcore"),
    compiler_params=pltpu.CompilerParams(needs_layout_passes=False),
    scratch_types=[
        pltpu.VMEM((N,), jnp.float32),
        pltpu.VMEM((N,), jnp.float32),
        pltpu.VMEM((N,), jnp.int32),
        pltpu.VMEM((8 * NC,), jnp.int32),
        pltpu.VMEM((NC + L,), jnp.int32),
        pltpu.VMEM((N,), jnp.int32),
        pltpu.VMEM((QPT, 2 * K), jnp.float32),
        pltpu.VMEM((NC,), jnp.int32),
        pltpu.SMEM((NC // L,), jnp.int32),
        pltpu.SemaphoreType.DMA,
        pltpu.SemaphoreType.DMA,
    ],
)


MLPR = 4096


@jax.jit
def kernel(xytp, W1, b1, W2, b2):
    px = xytp[..., 1]
    py = xytp[..., 2]
    emb, _, _ = _sc_ball(px, py, jnp.zeros((8 * NC,), jnp.int32))

    W1p = jnp.concatenate([W1[0::2, :], W1[1::2, :]], axis=0)
    out = pl.pallas_call(
        _mlp_kernel,
        grid=(B * N // MLPR,),
        in_specs=[
            pl.BlockSpec((MLPR, 2 * K), lambda i: (i, 0)),
            pl.BlockSpec((2 * K, 2 * OUT), lambda i: (0, 0)),
            pl.BlockSpec((1, 2 * OUT), lambda i: (0, 0)),
            pl.BlockSpec((2 * OUT, OUT), lambda i: (0, 0)),
            pl.BlockSpec((1, OUT), lambda i: (0, 0)),
        ],
        out_specs=pl.BlockSpec((MLPR, OUT), lambda i: (i, 0)),
        out_shape=jax.ShapeDtypeStruct((B * N, OUT), jnp.float32),
    )(emb.reshape(B * N, 2 * K), W1p, b1.reshape(1, -1), W2,
      b2.reshape(1, -1))
    return out.reshape(B, N, OUT)


# speedup vs baseline: 1.0376x; 1.0376x over previous
import functools

import jax
import jax.numpy as jnp
from jax import lax
from jax.experimental import pallas as pl
from jax.experimental.pallas import tpu as pltpu
from jax.experimental.pallas import tpu_sc as plsc

HEIGHT = 480
K = 16
OUT = 64
R2 = (5.0 / HEIGHT) ** 2

B = 4
N = 4096
G = 64
NC = G * G
BIG = 1 << 30
L = 16
QPT = N // 8


def _sc_ball_kernel(px_hbm, py_hbm, zeros_hbm, emb_hbm, start_hbm, order_hbm,
                    pxv, pyv, cellv, subh, startv, orderv, embv, sem1, sem2):
    c = lax.axis_index("core")
    s = lax.axis_index("subcore")
    lb = s // 8
    b = 2 * c + lb
    qs = (s % 8) * QPT

    lanes = lax.broadcasted_iota(jnp.int32, (L,), 0)
    ones = jnp.ones((L,), jnp.int32)

    cpx = pltpu.async_copy(px_hbm.at[b], pxv, sem1)
    cpy = pltpu.async_copy(py_hbm.at[b], pyv, sem2)
    cpx.wait()
    cpy.wait()

    @pl.when(s % 8 == 0)
    def _build():
        @pl.loop(0, N // L)
        def _cells(t):
            i0 = t * L
            x = pxv[pl.ds(i0, L)]
            y = pyv[pl.ds(i0, L)]
            cx = jnp.minimum((x * G).astype(jnp.int32), G - 1)
            cy = jnp.minimum((y * G).astype(jnp.int32), G - 1)
            cellv[pl.ds(i0, L)] = cy * G + cx

        pltpu.sync_copy(zeros_hbm, subh)

        mlow = lanes < 8
        mhigh = jnp.logical_not(mlow)
        rowbase = (lanes % 8) * NC

        @pl.loop(0, N // L)
        def _hist(t):
            cells16 = cellv[pl.ds(t * L, L)]
            flat = rowbase + cells16
            plsc.addupdate_scatter(subh, [flat], ones, mask=mlow)
            plsc.addupdate_scatter(subh, [flat], ones, mask=mhigh)

        def _ps(t, carry):
            c0 = t * L
            vs = [subh[pl.ds(l * NC + c0, L)] for l in range(8)]
            hist16 = vs[0]
            for l in range(1, 8):
                hist16 = hist16 + vs[l]
            incl = plsc.cumsum(hist16)
            start16 = incl - hist16 + carry
            startv[pl.ds(c0, L)] = start16
            run = start16
            for l in range(8):
                v = vs[l]
                subh[pl.ds(l * NC + c0, L)] = run
                run = run + v
            return carry + jnp.max(incl)

        lax.fori_loop(0, NC // L, _ps, jnp.int32(0))
        startv[pl.ds(NC, L)] = jnp.full((L,), N, jnp.int32)

        @pl.loop(0, N // L)
        def _scatter(t):
            i0 = t * L
            cells16 = cellv[pl.ds(i0, L)]
            flat = rowbase + cells16
            idxv = i0 + lanes
            pos1 = plsc.load_gather(subh, [flat], mask=mlow)
            plsc.addupdate_scatter(subh, [flat], ones, mask=mlow)
            plsc.store_scatter(orderv, [pos1], idxv, mask=mlow)
            pos2 = plsc.load_gather(subh, [flat], mask=mhigh)
            plsc.addupdate_scatter(subh, [flat], ones, mask=mhigh)
            plsc.store_scatter(orderv, [pos2], idxv, mask=mhigh)

        cps = pltpu.async_copy(startv, start_hbm.at[b], sem1)
        cpo = pltpu.async_copy(orderv, order_hbm.at[b], sem2)
        cps.wait()
        cpo.wait()

    plsc.subcore_barrier()

    cps = pltpu.async_copy(start_hbm.at[b], startv, sem1)
    cpo = pltpu.async_copy(order_hbm.at[b], orderv, sem2)
    cps.wait()
    cpo.wait()

    @pl.loop(0, QPT // L)
    def _group(g):
        q0 = qs + g * L
        qi = q0 + lanes
        qx = pxv[pl.ds(q0, L)]
        qy = pyv[pl.ds(q0, L)]
        cx = jnp.minimum((qx * G).astype(jnp.int32), G - 1)
        cy = jnp.minimum((qy * G).astype(jnp.int32), G - 1)
        c1 = jnp.maximum(cx - 1, 0)
        c2 = jnp.minimum(cx + 1, G - 1)

        slots = tuple(jnp.full((L,), BIG, jnp.int32) for _ in range(K))
        for dr in (-1, 0, 1):
            rr = cy + dr
            rvalid = (rr >= 0) & (rr < G)
            rrc = jnp.clip(rr, 0, G - 1)
            lo = plsc.load_gather(startv, [rrc * G + c1])
            hi = plsc.load_gather(startv, [rrc * G + c2 + 1])
            lo = jnp.clip(jnp.where(rvalid, lo, 0), 0, N)
            hi = jnp.clip(jnp.where(rvalid, hi, 0), 0, N)
            lenv = jnp.maximum(hi - lo, 0)
            maxlen = jnp.max(lenv)

            def _one(sl, t):
                active = t < lenv
                p = jnp.clip(jnp.where(active, lo + t, 0), 0, N - 1)
                j = jnp.clip(plsc.load_gather(orderv, [p]), 0, N - 1)
                x = plsc.load_gather(pxv, [j])
                y = plsc.load_gather(pyv, [j])
                dx = qx - x
                dy = qy - y
                qual = active & ((dx * dx + dy * dy) < R2)
                v = jnp.where(qual, j, BIG)
                ns = [jnp.minimum(sl[0], v)]
                for k in range(1, K):
                    ns.append(jnp.maximum(sl[k - 1], jnp.minimum(sl[k], v)))
                return tuple(ns)

            def _cand(t, sl):
                sl = _one(sl, 2 * t)
                return _one(sl, 2 * t + 1)

            slots = lax.fori_loop(0, (maxlen + 1) // 2, _cand, slots)

        rows = g * L + lanes
        for k in range(K):
            sk = slots[k]
            jk = jnp.where(sk < BIG, sk, qi)
            gx = plsc.load_gather(pxv, [jk])
            gy = plsc.load_gather(pyv, [jk])
            kcol = jnp.full((L,), k, jnp.int32)
            plsc.store_scatter(embv, [rows, kcol], qx - gx)
            plsc.store_scatter(embv, [rows, kcol + K], qy - gy)

    pltpu.sync_copy(embv, emb_hbm.at[b, pl.ds(qs, QPT)])


def _mlp_kernel(emb_ref, w1_ref, b1_ref, w2_ref, b2_ref, out_ref):
    e = emb_ref[...]
    h = lax.dot_general(e, w1_ref[...], (((1,), (0,)), ((), ())),
                        preferred_element_type=jnp.float32)
    h = jnp.maximum(h + b1_ref[...], 0.0)
    o = lax.dot_general(h, w2_ref[...], (((1,), (0,)), ((), ())),
                        preferred_element_type=jnp.float32)
    out_ref[...] = o + b2_ref[...]


_sc_ball = pl.kernel(
    _sc_ball_kernel,
    out_type=(jax.ShapeDtypeStruct((B, N, 2 * K), jnp.float32),
              jax.ShapeDtypeStruct((B, NC + L), jnp.int32),
              jax.ShapeDtypeStruct((B, N), jnp.int32)),
    mesh=plsc.VectorSubcoreMesh(core_axis_name="core",
                                subcore_axis_name="subcore"),
    compiler_params=pltpu.CompilerParams(needs_layout_passes=False),
    scratch_types=[
        pltpu.VMEM((N,), jnp.float32),
        pltpu.VMEM((N,), jnp.float32),
        pltpu.VMEM((N,), jnp.int32),
        pltpu.VMEM((8 * NC,), jnp.int32),
        pltpu.VMEM((NC + L,), jnp.int32),
        pltpu.VMEM((N,), jnp.int32),
        pltpu.VMEM((QPT, 2 * K), jnp.float32),
        pltpu.SemaphoreType.DMA,
        pltpu.SemaphoreType.DMA,
    ],
)


MLPR = 4096


@jax.jit
def kernel(xytp, W1, b1, W2, b2):
    px = xytp[..., 1]
    py = xytp[..., 2]
    emb, _, _ = _sc_ball(px, py, jnp.zeros((8 * NC,), jnp.int32))

    W1p = jnp.concatenate([W1[0::2, :], W1[1::2, :]], axis=0)
    out = pl.pallas_call(
        _mlp_kernel,
        grid=(B * N // MLPR,),
        in_specs=[
            pl.BlockSpec((MLPR, 2 * K), lambda i: (i, 0)),
            pl.BlockSpec((2 * K, 2 * OUT), lambda i: (0, 0)),
            pl.BlockSpec((1, 2 * OUT), lambda i: (0, 0)),
            pl.BlockSpec((2 * OUT, OUT), lambda i: (0, 0)),
            pl.BlockSpec((1, OUT), lambda i: (0, 0)),
        ],
        out_specs=pl.BlockSpec((MLPR, OUT), lambda i: (i, 0)),
        out_shape=jax.ShapeDtypeStruct((B * N, OUT), jnp.float32),
    )(emb.reshape(B * N, 2 * K), W1p, b1.reshape(1, -1), W2,
      b2.reshape(1, -1))
    return out.reshape(B, N, OUT)
